# baseline (speedup 1.0000x reference)
import jax
import jax.numpy as jnp
from jax.experimental import pallas as pl

_L_BLK = 4


def _concat_body(x_ref, ep_ref, o_ref):
    o_ref[:, :64, :] = x_ref[...]
    ep = ep_ref[...]
    o_ref[:, 64:, :] = jnp.tile(ep, (1, 1, 32))


def kernel(x, table):
    B, L, D = x.shape
    E = table.shape[-1]
    W = D + E

    xt = jnp.transpose(x, (1, 2, 0))
    ep = jnp.broadcast_to(table[:, :, None], (L, E, 128))

    out_t = pl.pallas_call(
        _concat_body,
        grid=(L // _L_BLK,),
        in_specs=[
            pl.BlockSpec((_L_BLK, D, B), lambda i: (i, 0, 0)),
            pl.BlockSpec((_L_BLK, E, 128), lambda i: (i, 0, 0)),
        ],
        out_specs=pl.BlockSpec((_L_BLK, W, B), lambda i: (i, 0, 0)),
        out_shape=jax.ShapeDtypeStruct((L, W, B), x.dtype),
    )(xt, ep)
    return jnp.transpose(out_t, (2, 0, 1))


# speedup vs baseline: 1.0219x; 1.0219x over previous
import jax
import jax.numpy as jnp
from jax.experimental import pallas as pl

_L_BLK = 10


def _concat_body(x_ref, ep_ref, o_ref):
    o_ref[:, :64, :] = x_ref[...]
    ep = ep_ref[...]
    o_ref[:, 64:, :] = jnp.tile(ep, (1, 1, 32))


def kernel(x, table):
    B, L, D = x.shape
    E = table.shape[-1]
    W = D + E

    xt = jnp.transpose(x, (1, 2, 0))
    ep = jnp.broadcast_to(table[:, :, None], (L, E, 128))

    out_t = pl.pallas_call(
        _concat_body,
        grid=(L // _L_BLK,),
        in_specs=[
            pl.BlockSpec((_L_BLK, D, B), lambda i: (i, 0, 0)),
            pl.BlockSpec((_L_BLK, E, 128), lambda i: (i, 0, 0)),
        ],
        out_specs=pl.BlockSpec((_L_BLK, W, B), lambda i: (i, 0, 0)),
        out_shape=jax.ShapeDtypeStruct((L, W, B), x.dtype),
    )(xt, ep)
    return jnp.transpose(out_t, (2, 0, 1))
